# Initial kernel scaffold; baseline (speedup 1.0000x reference)
import jax
import jax.numpy as jnp
from jax.experimental import pallas as pl


def kernel(x, fc1_w, fc1_b, fc2_w, fc2_b, fc3_w, fc3_b, gcn_w1, gcn_b1, gcn_w2, gcn_b2, mlp_w1, mlp_b1, mlp_w2):
    raise NotImplementedError("write your pallas kernel here")


# speedup vs baseline: 4.9538x; 4.9538x over previous
import jax
import jax.numpy as jnp
from jax import lax
from jax.experimental import pallas as pl

N, D, HID, OUT = 4096, 512, 128, 512
EPS = 2.220446049250313e-16
BM = 256
BM1 = 512


def _bdot(a, b):
    return jnp.dot(a.astype(jnp.bfloat16), b.astype(jnp.bfloat16),
                   preferred_element_type=jnp.float32)


def _emb_kernel(x_ref, w1_ref, b1_ref, w2_ref, b2_ref, w3_ref, b3_ref,
                emb_ref, sq_ref):
    x = x_ref[...]
    h = jnp.maximum(_bdot(x, w1_ref[...]) + b1_ref[...][None, :], 0.0)
    h = jnp.maximum(_bdot(h, w2_ref[...]) + b2_ref[...][None, :], 0.0)
    sigma = _bdot(h, w3_ref[...]) + b3_ref[...][None, :]
    emb = x / (sigma + EPS)
    emb_ref[...] = emb
    sq_ref[...] = jnp.sum(emb * emb, axis=1)


def _topk_kernel(embb_ref, emba_ref, sqc_ref, w_ref, topi_ref, topv_ref):
    embb = embb_ref[...]
    emba = emba_ref[...]
    sqr = jnp.sum(embb * embb, axis=1, keepdims=True)
    sqc = sqc_ref[...][None, :]
    g = lax.dot_general(embb.astype(jnp.bfloat16), emba.astype(jnp.bfloat16),
                        (((1,), (1,)), ((), ())),
                        preferred_element_type=jnp.float32)
    w = jnp.exp(-((sqr + sqc - 2.0 * g) / D) / 2.0)
    w_ref[...] = w
    iota = lax.broadcasted_iota(jnp.int32, (BM, N), 1)
    cur = w
    vals, idxs = [], []
    for _ in range(3):
        m = jnp.max(cur, axis=1, keepdims=True)
        sel = jnp.min(jnp.where(cur == m, iota, N), axis=1, keepdims=True)
        vals.append(m)
        idxs.append(sel)
        cur = jnp.where(iota == sel, -jnp.inf, cur)
    topv_ref[...] = jnp.concatenate(vals + [jnp.zeros_like(vals[0])], axis=1)
    topi_ref[...] = jnp.concatenate(idxs + [jnp.zeros_like(idxs[0])], axis=1)


def _adj_tile(w, topir, topit, blk):
    iota_c = lax.broadcasted_iota(jnp.int32, (BM, N), 1)
    ig = lax.broadcasted_iota(jnp.int32, (BM, 1), 0) + blk * BM
    outm = jnp.zeros((BM, N), jnp.bool_)
    inm = jnp.zeros((BM, N), jnp.bool_)
    for k in range(3):
        outm = outm | (topir[:, k:k + 1] == iota_c)
        inm = inm | (topit[k:k + 1, :] == ig)
    return ((outm | inm) & (w > 0.0)).astype(jnp.float32)


def _deg_kernel(w_ref, topir_ref, topit_ref, deg_ref):
    blk = pl.program_id(0)
    a = _adj_tile(w_ref[...], topir_ref[...], topit_ref[...], blk)
    deg_ref[...] = jnp.sum(a, axis=1, keepdims=True)


def _dinv_kernel(deg_ref, dinv_ref):
    deg = jnp.maximum(deg_ref[...], 1.0)
    dinv_ref[...] = 1.0 / jnp.sqrt(deg)


def _gcn1_kernel(w_ref, topir_ref, topit_ref, dinvb_ref, dinvr_ref,
                 x_ref, gw1_ref, gb1_ref, h1_ref):
    blk = pl.program_id(0)
    a = _adj_tile(w_ref[...], topir_ref[...], topit_ref[...], blk)
    da = dinvb_ref[...] * a * dinvr_ref[...]
    agg = lax.dot_general(da, x_ref[...], (((1,), (0,)), ((), ())))
    z = jnp.dot(agg, gw1_ref[...]) + gb1_ref[...][None, :]
    h1_ref[...] = jnp.where(z > 0.0, z, jnp.exp(z) - 1.0)


def _gcn2_kernel(w_ref, topir_ref, topit_ref, dinvb_ref, dinvr_ref,
                 h1_ref, xb_ref, gw2_ref, gb2_ref, mw1_ref, mb1_ref, mw2_ref,
                 out_ref):
    blk = pl.program_id(0)
    a = _adj_tile(w_ref[...], topir_ref[...], topit_ref[...], blk)
    da = dinvb_ref[...] * a * dinvr_ref[...]
    agg = lax.dot_general(da, h1_ref[...], (((1,), (0,)), ((), ())))
    z = jnp.dot(agg, gw2_ref[...]) + gb2_ref[...][None, :]
    h2 = jnp.where(z > 0.0, z, jnp.exp(z) - 1.0)
    g = h2 + xb_ref[...]
    m = jnp.maximum(jnp.dot(g, mw1_ref[...]) + mb1_ref[...][None, :], 0.0)
    out_ref[...] = jnp.dot(m, mw2_ref[...])


def _full(shape):
    return pl.BlockSpec(shape, lambda i: (0,) * len(shape))


def _rows(b, width):
    return pl.BlockSpec((b, width), lambda i: (i, 0))


def kernel(x, fc1_w, fc1_b, fc2_w, fc2_b, fc3_w, fc3_b,
           gcn_w1, gcn_b1, gcn_w2, gcn_b2, mlp_w1, mlp_b1, mlp_w2):
    emb, sq = pl.pallas_call(
        _emb_kernel,
        grid=(N // BM1,),
        in_specs=[_rows(BM1, D), _full((D, 64)), _full((64,)),
                  _full((64, 16)), _full((16,)), _full((16, 1)), _full((1,))],
        out_specs=[_rows(BM1, D), pl.BlockSpec((BM1,), lambda i: (i,))],
        out_shape=[jax.ShapeDtypeStruct((N, D), jnp.float32),
                   jax.ShapeDtypeStruct((N,), jnp.float32)],
    )(x, fc1_w, fc1_b, fc2_w, fc2_b, fc3_w, fc3_b)

    wfull, topi, topv = pl.pallas_call(
        _topk_kernel,
        grid=(N // BM,),
        in_specs=[_rows(BM, D), _full((N, D)), _full((N,))],
        out_specs=[_rows(BM, N), _rows(BM, 4), _rows(BM, 4)],
        out_shape=[jax.ShapeDtypeStruct((N, N), jnp.float32),
                   jax.ShapeDtypeStruct((N, 4), jnp.int32),
                   jax.ShapeDtypeStruct((N, 4), jnp.float32)],
    )(emb, emb, sq)

    topit = topi.T
    deg = pl.pallas_call(
        _deg_kernel,
        grid=(N // BM,),
        in_specs=[_rows(BM, N), _rows(BM, 4), _full((4, N))],
        out_specs=_rows(BM, 1),
        out_shape=jax.ShapeDtypeStruct((N, 1), jnp.float32),
    )(wfull, topi, topit)

    dinv = pl.pallas_call(
        _dinv_kernel,
        grid=(1,),
        in_specs=[_full((N, 1))],
        out_specs=_full((N, 1)),
        out_shape=jax.ShapeDtypeStruct((N, 1), jnp.float32),
    )(deg)
    dinvr = dinv.reshape(1, N)

    h1 = pl.pallas_call(
        _gcn1_kernel,
        grid=(N // BM,),
        in_specs=[_rows(BM, N), _rows(BM, 4), _full((4, N)), _rows(BM, 1),
                  _full((1, N)), _full((N, D)), _full((D, HID)), _full((HID,))],
        out_specs=_rows(BM, HID),
        out_shape=jax.ShapeDtypeStruct((N, HID), jnp.float32),
    )(wfull, topi, topit, dinv, dinvr, x, gcn_w1, gcn_b1)

    out = pl.pallas_call(
        _gcn2_kernel,
        grid=(N // BM,),
        in_specs=[_rows(BM, N), _rows(BM, 4), _full((4, N)), _rows(BM, 1),
                  _full((1, N)), _full((N, HID)), _rows(BM, D),
                  _full((HID, OUT)), _full((OUT,)),
                  _full((OUT, HID)), _full((HID,)), _full((HID, OUT))],
        out_specs=_rows(BM, OUT),
        out_shape=jax.ShapeDtypeStruct((N, OUT), jnp.float32),
    )(wfull, topi, topit, dinv, dinvr, h1, x,
      gcn_w2, gcn_b2, mlp_w1, mlp_b1, mlp_w2)
    return out
